# baseline (speedup 1.0000x reference)
import functools
import math

import jax
import jax.numpy as jnp
from jax import lax
from jax.experimental import pallas as pl
from jax.experimental.pallas import tpu as pltpu
from jax.experimental.pallas import tpu_sc as plsc

N = 10000
E = 320000
H = 128
OUT = 7
P = 500000
B = 16384
ALPHA = 0.1
EPS = 1e-5

NPAD = 10240
EPAD = 327680
CH = 128
ECH = EPAD // CH
RPT = NPAD // 16
ZR = 64

@functools.lru_cache(maxsize=None)
def _mesh():
    return plsc.VectorSubcoreMesh(core_axis_name="c", subcore_axis_name="s")


@functools.lru_cache(maxsize=None)
def _build_deg():
    return functools.partial(
        pl.kernel,
        out_type=jax.ShapeDtypeStruct((2, NPAD), jnp.float32),
        mesh=_mesh(),
        scratch_types=[
            pltpu.VMEM((ECH // 16, CH), jnp.int32),
            pltpu.VMEM((CH,), jnp.float32),
            pltpu.VMEM((RPT,), jnp.float32),
            pltpu.VMEM_SHARED((NPAD,), jnp.float32),
            pltpu.SemaphoreType.DMA,
            pltpu.SemaphoreType.DMA,
            pltpu.SemaphoreType.DMA,
            pltpu.SemaphoreType.DMA,
        ],
    )(_deg_body)


def _deg_body(eidx_hbm, out_hbm, idx_v, ones_v, stage_v, hist_sp,
              m0, m1, m2, m3):
    c = lax.axis_index("c")
    s = lax.axis_index("s")
    sems = (m0, m1, m2, m3)
    one = jnp.ones((16,), jnp.float32)
    zero = jnp.zeros((16,), jnp.float32)

    def fill(k, carry):
        ones_v[pl.ds(k * 16, 16)] = one
        return carry

    lax.fori_loop(0, CH // 16, fill, 0)

    def zstage(k, carry):
        stage_v[pl.ds(k * 16, 16)] = zero
        return carry

    lax.fori_loop(0, RPT // 16, zstage, 0)
    pltpu.sync_copy(stage_v, hist_sp.at[pl.ds(s * RPT, RPT)])
    plsc.subcore_barrier()

    nch = ECH // 16
    pltpu.sync_copy(eidx_hbm.at[c].at[pl.ds(s * nch, nch)], idx_v)

    for b in range(4):
        pltpu.async_copy(ones_v, hist_sp.at[idx_v.at[b]], sems[b],
                         add=True)

    def step(t, carry):
        j = 4 * t
        for b in range(4):
            jj4 = j + 4 + b

            @pl.when(jj4 < nch)
            def _(b=b, jj4=jj4):
                pltpu.make_async_copy(
                    ones_v, hist_sp.at[idx_v.at[jj4 - 4]], sems[b]).wait()
                pltpu.async_copy(ones_v, hist_sp.at[idx_v.at[jj4]],
                                 sems[b], add=True)
        return carry

    lax.fori_loop(0, nch // 4, step, 0)
    for b in range(4):
        pltpu.make_async_copy(
            ones_v, hist_sp.at[idx_v.at[nch - 4 + b]], sems[b]).wait()
    plsc.subcore_barrier()

    pltpu.sync_copy(hist_sp.at[pl.ds(s * RPT, RPT)], stage_v)
    pltpu.sync_copy(stage_v, out_hbm.at[c].at[pl.ds(s * RPT, RPT)])


QH = H // 4
MCH = 256
_MPN = EPAD // 16 // MCH


@functools.lru_cache(maxsize=None)
def _build_mp(ntab):
    @functools.partial(
        pl.kernel,
        out_type=(jax.ShapeDtypeStruct((4, NPAD, QH), jnp.float32),) * ntab,
        mesh=_mesh(),
        compiler_params=pltpu.CompilerParams(use_tc_tiling_on_sc=False),
        scratch_types=[
            pltpu.VMEM((_MPN, MCH), jnp.int32),
            pltpu.VMEM((_MPN, MCH), jnp.int32),
            pltpu.VMEM((MCH, QH), jnp.float32),
            pltpu.VMEM((MCH, QH), jnp.float32),
            pltpu.VMEM((RPT, QH), jnp.float32),
            pltpu.VMEM_SHARED((NPAD, QH), jnp.float32),
            pltpu.VMEM_SHARED((NPAD, QH), jnp.float32),
            pltpu.SemaphoreType.DMA,
            pltpu.SemaphoreType.DMA,
            pltpu.SemaphoreType.DMA,
            pltpu.SemaphoreType.DMA,
        ],
    )
    def mp(*args):
        tabs = args[:ntab]
        src_hbm, dst_hbm = args[ntab:ntab + 2]
        outs = args[ntab + 2:2 * ntab + 2]
        (si, di, r0, r1, zb, tbl, agg, g0, g1, s0, s1) = args[2 * ntab + 2:]
        c = lax.axis_index("c")
        s = lax.axis_index("s")
        bufs = (r0, r1)
        gsems = (g0, g1)
        ssems = (s0, s1)

        zv = jnp.zeros((16,), jnp.float32)

        base = s * _MPN
        pltpu.sync_copy(src_hbm.at[pl.ds(base, _MPN)], si)
        pltpu.sync_copy(dst_hbm.at[pl.ds(base, _MPN)], di)

        for xs_hbm, out_hbm, q in [(tabs[t], outs[t], qq)
                                   for t in range(ntab) for qq in range(2)]:
            quarter = 2 * c + q
            pltpu.sync_copy(xs_hbm.at[quarter].at[pl.ds(s * RPT, RPT)],
                            tbl.at[pl.ds(s * RPT, RPT)])

            def zs(k, carry):
                r = k // 2
                col = (k % 2) * 16
                zb[r, pl.ds(col, 16)] = zv
                return carry

            lax.fori_loop(0, RPT * 2, zs, 0)
            pltpu.sync_copy(zb, agg.at[pl.ds(s * RPT, RPT)])
            plsc.subcore_barrier()

            for b in range(2):
                pltpu.async_copy(tbl.at[si.at[b]], bufs[b], gsems[b])

            def step(t, carry):
                j = 2 * t
                for b in range(2):
                    jj = j + b
                    pltpu.make_async_copy(
                        tbl.at[si.at[jj]], bufs[b], gsems[b]).wait()
                    pltpu.async_copy(bufs[b], agg.at[di.at[jj]], ssems[b],
                                     add=True)
                for b in range(2):
                    jj2 = j + 2 + b

                    @pl.when(jj2 < _MPN)
                    def _(b=b, jj2=jj2):
                        pltpu.make_async_copy(
                            bufs[b], agg.at[di.at[jj2 - 2]],
                            ssems[b]).wait()
                        pltpu.async_copy(tbl.at[si.at[jj2]], bufs[b],
                                         gsems[b])
                return carry

            lax.fori_loop(0, _MPN // 2, step, 0)
            for b in range(2):
                pltpu.make_async_copy(
                    bufs[b], agg.at[di.at[_MPN - 2 + b]], ssems[b]).wait()
            plsc.subcore_barrier()

            off = s * RPT
            pltpu.sync_copy(agg.at[pl.ds(off, RPT)], zb)
            pltpu.sync_copy(zb, out_hbm.at[quarter].at[pl.ds(off, RPT)])
            plsc.subcore_barrier()

    return mp


_PW = B // 32
_PCH = _PW // CH


@functools.lru_cache(maxsize=None)
def _build_ppi():
    return functools.partial(
        pl.kernel,
        out_type=(
            jax.ShapeDtypeStruct((B, H), jnp.float32),
            jax.ShapeDtypeStruct((B, H), jnp.float32),
        ),
        mesh=_mesh(),
        scratch_types=[
            pltpu.VMEM((_PCH, CH), jnp.int32),
            pltpu.VMEM((_PCH, CH), jnp.int32),
            pltpu.VMEM((_PCH, CH), jnp.int32),
            pltpu.VMEM((CH, H), jnp.float32),
            pltpu.VMEM((CH, H), jnp.float32),
            pltpu.VMEM((CH, H), jnp.float32),
            pltpu.VMEM((CH, H), jnp.float32),
            pltpu.SemaphoreType.DMA,
            pltpu.SemaphoreType.DMA,
            pltpu.SemaphoreType.DMA,
            pltpu.SemaphoreType.DMA,
        ],
    )(_ppi_body)


def _ppi_body(ppi0_hbm, ppi1_hbm, idx_hbm, xenc_hbm, menc_hbm,
              outx_hbm, outm_hbm, it, n0, n1, xa, xb, ma, mb,
              m0, m1, m2, m3):
    c = lax.axis_index("c")
    s = lax.axis_index("s")
    w = c * 16 + s

    pltpu.sync_copy(idx_hbm.at[pl.ds(w * _PCH, _PCH)], it)

    def chunk(j, carry):
        pltpu.async_copy(ppi0_hbm.at[it.at[j]], n0.at[j], m0)
        pltpu.async_copy(ppi1_hbm.at[it.at[j]], n1.at[j], m1)
        pltpu.make_async_copy(ppi0_hbm.at[it.at[j]], n0.at[j], m0).wait()
        pltpu.make_async_copy(ppi1_hbm.at[it.at[j]], n1.at[j], m1).wait()

        pltpu.async_copy(xenc_hbm.at[n0.at[j]], xa, m0)
        pltpu.async_copy(xenc_hbm.at[n1.at[j]], xb, m1)
        pltpu.async_copy(menc_hbm.at[n0.at[j]], ma, m2)
        pltpu.async_copy(menc_hbm.at[n1.at[j]], mb, m3)

        def pair(ra, rb, out_hbm, sa, sb, src_a, src_b):
            pltpu.make_async_copy(src_a, ra, sa).wait()
            pltpu.make_async_copy(src_b, rb, sb).wait()

            def mul(k, cc):
                r = k // 8
                col = (k % 8) * 16
                ra[r, pl.ds(col, 16)] = (
                    ra[r, pl.ds(col, 16)] * rb[r, pl.ds(col, 16)]
                )
                return cc

            lax.fori_loop(0, CH * 8, mul, 0)
            pltpu.sync_copy(ra, out_hbm.at[pl.ds(w * _PW + j * CH, CH)])

        pair(xa, xb, outx_hbm, m0, m1,
             xenc_hbm.at[n0.at[j]], xenc_hbm.at[n1.at[j]])
        pair(ma, mb, outm_hbm, m2, m3,
             menc_hbm.at[n0.at[j]], menc_hbm.at[n1.at[j]])
        return carry

    lax.fori_loop(0, _PCH, chunk, 0)


def _tc_input(x, mask, w, b2, g2, bb2):
    def body(x_ref, m_ref, w_ref, b_ref, g_ref, bb_ref, o1_ref, o2_ref):
        h = jnp.dot(x_ref[...], w_ref[...],
                    preferred_element_type=jnp.float32)
        h = (h + b_ref[...]) * g_ref[...] + bb_ref[...]
        o1_ref[...] = h
        o2_ref[...] = h * m_ref[...]

    blk = 2000
    grid = N // blk
    return pl.pallas_call(
        body,
        grid=(grid,),
        in_specs=[
            pl.BlockSpec((blk, H), lambda i: (i, 0)),
            pl.BlockSpec((blk, H), lambda i: (i, 0)),
            pl.BlockSpec((H, H), lambda i: (0, 0)),
            pl.BlockSpec((1, H), lambda i: (0, 0)),
            pl.BlockSpec((1, H), lambda i: (0, 0)),
            pl.BlockSpec((1, H), lambda i: (0, 0)),
        ],
        out_specs=[
            pl.BlockSpec((blk, H), lambda i: (i, 0)),
            pl.BlockSpec((blk, H), lambda i: (i, 0)),
        ],
        out_shape=[
            jax.ShapeDtypeStruct((N, H), jnp.float32),
            jax.ShapeDtypeStruct((N, H), jnp.float32),
        ],
    )(x, mask, w, b2, g2, bb2)


def _split(ref, v):
    for q in range(4):
        ref[q] = v[:, q * QH:(q + 1) * QH]


def _tc_mkxs(deg, xbn_p, mx_p):
    def body(d_ref, a_ref, m_ref, xsa_ref, xsb_ref, x0_ref, nsd_ref):
        d = d_ref[...]
        nrm = jnp.where(
            d > 0.0, lax.rsqrt(jnp.maximum(d, 1e-12)), 0.0)
        nsd_ref[...] = nrm
        ns = nrm[0][:, None]
        a = a_ref[...]
        m = m_ref[...]
        _split(xsa_ref, a * ns)
        _split(xsb_ref, m * ns)
        x0_ref[0] = a
        x0_ref[1] = m

    blk = 1280
    grid = NPAD // blk
    return pl.pallas_call(
        body,
        grid=(grid,),
        in_specs=[
            pl.BlockSpec((2, blk), lambda i: (0, i)),
            pl.BlockSpec((blk, H), lambda i: (i, 0)),
            pl.BlockSpec((blk, H), lambda i: (i, 0)),
        ],
        out_specs=[
            pl.BlockSpec((4, blk, QH), lambda i: (0, i, 0)),
            pl.BlockSpec((4, blk, QH), lambda i: (0, i, 0)),
            pl.BlockSpec((2, blk, H), lambda i: (0, i, 0)),
            pl.BlockSpec((2, blk), lambda i: (0, i)),
        ],
        out_shape=[
            jax.ShapeDtypeStruct((4, NPAD, QH), jnp.float32),
            jax.ShapeDtypeStruct((4, NPAD, QH), jnp.float32),
            jax.ShapeDtypeStruct((2, NPAD, H), jnp.float32),
            jax.ShapeDtypeStruct((2, NPAD), jnp.float32),
        ],
    )(deg, xbn_p, mx_p)


def _gcn_combine(agg, nd, x0, w_ref, beta):
    feat = (1.0 - ALPHA) * (agg * nd) + ALPHA * x0
    return (1.0 - beta) * feat + beta * jnp.dot(
        feat, w_ref[...], preferred_element_type=jnp.float32)


def _join(a_ref):
    return jnp.concatenate([a_ref[0], a_ref[1], a_ref[2], a_ref[3]],
                           axis=-1)


def _tc_layer1_enc(agg_a, agg_b, nsd, x0_pair, w):
    beta = math.log(2.0)

    def body(aa_ref, ab_ref, n_ref, x0_ref, w_ref, oa_ref, ob_ref):
        nrm = n_ref[...]
        nd = nrm[1][:, None]
        ns = nrm[0][:, None]
        for a_ref, x0, o_ref in ((aa_ref, x0_ref[0], oa_ref),
                                 (ab_ref, x0_ref[1], ob_ref)):
            h = _gcn_combine(_join(a_ref), nd, x0, w_ref, beta)
            _split(o_ref, h * ns)

    blk = 1280
    grid = NPAD // blk
    return pl.pallas_call(
        body,
        grid=(grid,),
        in_specs=[
            pl.BlockSpec((4, blk, QH), lambda i: (0, i, 0)),
            pl.BlockSpec((4, blk, QH), lambda i: (0, i, 0)),
            pl.BlockSpec((2, blk), lambda i: (0, i)),
            pl.BlockSpec((2, blk, H), lambda i: (0, i, 0)),
            pl.BlockSpec((H, H), lambda i: (0, 0)),
        ],
        out_specs=[
            pl.BlockSpec((4, blk, QH), lambda i: (0, i, 0)),
            pl.BlockSpec((4, blk, QH), lambda i: (0, i, 0)),
        ],
        out_shape=[
            jax.ShapeDtypeStruct((4, NPAD, QH), jnp.float32),
            jax.ShapeDtypeStruct((4, NPAD, QH), jnp.float32),
        ],
    )(agg_a, agg_b, nsd, x0_pair, w)


def _tc_layer2_enc(agg_a, agg_b, nsd, x0_pair, w, fcw, fcb, g2, bb2,
                   fc2w, fc2b):
    beta = math.log(1.5)

    def body(aa_ref, ab_ref, n_ref, x0_ref, w_ref, fcw_ref, fcb_ref, g_ref,
             bb_ref, fc2w_ref, fc2b_ref, enc_ref, xsd_ref):
        nrm = n_ref[...]
        nd = nrm[1][:, None]
        ns = nrm[0][:, None]
        for t, a_ref in ((0, aa_ref), (1, ab_ref)):
            h = _gcn_combine(_join(a_ref), nd, x0_ref[t], w_ref, beta)
            u = jax.nn.relu(jnp.dot(h, fcw_ref[...],
                                    preferred_element_type=jnp.float32)
                            + fcb_ref[...])
            u = u * g_ref[...] + bb_ref[...]
            v = jax.nn.relu(jnp.dot(u, fc2w_ref[...],
                                    preferred_element_type=jnp.float32)
                            + fc2b_ref[...])
            enc_ref[t] = v
            if t == 0:
                _split(xsd_ref, v * ns)

    blk = 1280
    grid = NPAD // blk
    return pl.pallas_call(
        body,
        grid=(grid,),
        in_specs=[
            pl.BlockSpec((4, blk, QH), lambda i: (0, i, 0)),
            pl.BlockSpec((4, blk, QH), lambda i: (0, i, 0)),
            pl.BlockSpec((2, blk), lambda i: (0, i)),
            pl.BlockSpec((2, blk, H), lambda i: (0, i, 0)),
            pl.BlockSpec((H, H), lambda i: (0, 0)),
            pl.BlockSpec((H, 2 * H), lambda i: (0, 0)),
            pl.BlockSpec((1, 2 * H), lambda i: (0, 0)),
            pl.BlockSpec((1, 2 * H), lambda i: (0, 0)),
            pl.BlockSpec((1, 2 * H), lambda i: (0, 0)),
            pl.BlockSpec((2 * H, H), lambda i: (0, 0)),
            pl.BlockSpec((1, H), lambda i: (0, 0)),
        ],
        out_specs=[
            pl.BlockSpec((2, blk, H), lambda i: (0, i, 0)),
            pl.BlockSpec((4, blk, QH), lambda i: (0, i, 0)),
        ],
        out_shape=[
            jax.ShapeDtypeStruct((2, NPAD, H), jnp.float32),
            jax.ShapeDtypeStruct((4, NPAD, QH), jnp.float32),
        ],
    )(agg_a, agg_b, nsd, x0_pair, w, fcw, fcb, g2, bb2, fc2w, fc2b)


def _tc_layer1_dec(agg, nsd, x0, w):
    beta = math.log(2.0)

    def body(a_ref, n_ref, x0_ref, w_ref, o_ref):
        nrm = n_ref[...]
        h = _gcn_combine(_join(a_ref), nrm[1][:, None], x0_ref[...],
                         w_ref, beta)
        _split(o_ref, h * nrm[0][:, None])

    blk = 1280
    grid = NPAD // blk
    return pl.pallas_call(
        body,
        grid=(grid,),
        in_specs=[
            pl.BlockSpec((4, blk, QH), lambda i: (0, i, 0)),
            pl.BlockSpec((2, blk), lambda i: (0, i)),
            pl.BlockSpec((blk, H), lambda i: (i, 0)),
            pl.BlockSpec((H, H), lambda i: (0, 0)),
        ],
        out_specs=pl.BlockSpec((4, blk, QH), lambda i: (0, i, 0)),
        out_shape=jax.ShapeDtypeStruct((4, NPAD, QH), jnp.float32),
    )(agg, nsd, x0, w)


def _tc_layer2_dec(agg, nsd, x0, w, fcw, fcb, g2, bb2, fc2w, fc2b):
    beta = math.log(1.5)

    def body(a_ref, n_ref, x0_ref, w_ref, fcw_ref, fcb_ref, g_ref, bb_ref,
             fc2w_ref, fc2b_ref, o_ref):
        nrm = n_ref[...]
        h = _gcn_combine(_join(a_ref), nrm[1][:, None], x0_ref[...],
                         w_ref, beta)
        u = jax.nn.relu(jnp.dot(h, fcw_ref[...],
                                preferred_element_type=jnp.float32)
                        + fcb_ref[...])
        u = u * g_ref[...] + bb_ref[...]
        o_ref[...] = jax.nn.relu(jnp.dot(u, fc2w_ref[...],
                                         preferred_element_type=jnp.float32)
                                 + fc2b_ref[...])

    blk = 1280
    grid = NPAD // blk
    return pl.pallas_call(
        body,
        grid=(grid,),
        in_specs=[
            pl.BlockSpec((4, blk, QH), lambda i: (0, i, 0)),
            pl.BlockSpec((2, blk), lambda i: (0, i)),
            pl.BlockSpec((blk, H), lambda i: (i, 0)),
            pl.BlockSpec((H, H), lambda i: (0, 0)),
            pl.BlockSpec((H, 2 * H), lambda i: (0, 0)),
            pl.BlockSpec((1, 2 * H), lambda i: (0, 0)),
            pl.BlockSpec((1, 2 * H), lambda i: (0, 0)),
            pl.BlockSpec((1, 2 * H), lambda i: (0, 0)),
            pl.BlockSpec((2 * H, H), lambda i: (0, 0)),
            pl.BlockSpec((1, H), lambda i: (0, 0)),
        ],
        out_specs=pl.BlockSpec((blk, H), lambda i: (i, 0)),
        out_shape=jax.ShapeDtypeStruct((NPAD, H), jnp.float32),
    )(agg, nsd, x0, w, fcw, fcb, g2, bb2, fc2w, fc2b)


def _tc_loss(recon_p, xbn_p):
    def body(r_ref, x_ref, o_ref):
        i = pl.program_id(0)

        @pl.when(i == 0)
        def _():
            o_ref[...] = jnp.zeros((1, 1), jnp.float32)

        rows = lax.broadcasted_iota(jnp.int32, (1280, H), 0) + i * 1280
        m = (rows < N).astype(jnp.float32)
        d = (r_ref[...] - x_ref[...]) * m
        o_ref[...] += jnp.sum(d * d).reshape(1, 1) / (N * H)

    return pl.pallas_call(
        body,
        grid=(NPAD // 1280,),
        in_specs=[
            pl.BlockSpec((1280, H), lambda i: (i, 0)),
            pl.BlockSpec((1280, H), lambda i: (i, 0)),
        ],
        out_specs=pl.BlockSpec((1, 1), lambda i: (0, 0)),
        out_shape=jax.ShapeDtypeStruct((1, 1), jnp.float32),
    )(recon_p, xbn_p)


def _tc_head(prod, w, b2):
    def body(p_ref, w_ref, b_ref, o_ref):
        o_ref[...] = jnp.dot(p_ref[...], w_ref[...],
                             preferred_element_type=jnp.float32) + b_ref[...]

    blk = 2048
    return pl.pallas_call(
        body,
        grid=(B // blk,),
        in_specs=[
            pl.BlockSpec((blk, H), lambda i: (i, 0)),
            pl.BlockSpec((H, OUT), lambda i: (0, 0)),
            pl.BlockSpec((1, OUT), lambda i: (0, 0)),
        ],
        out_specs=pl.BlockSpec((blk, OUT), lambda i: (i, 0)),
        out_shape=jax.ShapeDtypeStruct((B, OUT), jnp.float32),
    )(prod, w, b2)


def kernel(x, edge_index, ppi_list, idx, mask, fc_dim_W, fc_dim_b,
           norm_in_g, norm_in_b, enc_gcn_W, enc_fc_W, enc_fc_b, enc_norm_g,
           enc_norm_b, enc_fc2_W, enc_fc2_b, dec_gcn_W, dec_fc_W, dec_fc_b,
           dec_norm_g, dec_norm_b, dec_fc2_W, dec_fc2_b, fc_out_W, fc_out_b):
    f32 = jnp.float32
    i32 = jnp.int32
    inv = 1.0 / math.sqrt(1.0 + EPS)

    src = edge_index[0]
    dst = edge_index[1]
    pad = jnp.full((EPAD - E,), N, i32)
    srcf = jnp.concatenate([src, pad])
    dstf = jnp.concatenate([dst, pad])
    eidx = jnp.stack([srcf.reshape(ECH, CH), dstf.reshape(ECH, CH)])
    srcp = srcf.reshape(EPAD // MCH, MCH)
    dstp = dstf.reshape(EPAD // MCH, MCH)

    g_in2 = (norm_in_g * inv).reshape(1, H)
    b_in2 = norm_in_b.reshape(1, H)
    fcb2 = fc_dim_b.reshape(1, H)
    eg2 = (enc_norm_g * inv).reshape(1, 2 * H)
    eb2 = enc_norm_b.reshape(1, 2 * H)
    dg2 = (dec_norm_g * inv).reshape(1, 2 * H)
    db2 = dec_norm_b.reshape(1, 2 * H)

    deg = _build_deg()(eidx)
    xbn, mx = _tc_input(x, mask, fc_dim_W, fcb2, g_in2, b_in2)
    xbn_p = jnp.pad(xbn, ((0, NPAD - N), (0, 0)))
    mx_p = jnp.pad(mx, ((0, NPAD - N), (0, 0)))

    xsa, xsb, x0_pair, nsd = _tc_mkxs(deg, xbn_p, mx_p)

    mp2 = _build_mp(2)
    mp1 = _build_mp(1)
    agg1a, agg1b = mp2(xsa, xsb, srcp, dstp)
    xs1a, xs1b = _tc_layer1_enc(agg1a, agg1b, nsd, x0_pair, enc_gcn_W[0])
    agg2a, agg2b = mp2(xs1a, xs1b, srcp, dstp)
    enc_pair, xsd = _tc_layer2_enc(
        agg2a, agg2b, nsd, x0_pair, enc_gcn_W[1], enc_fc_W,
        enc_fc_b.reshape(1, -1), eg2, eb2, enc_fc2_W, enc_fc2_b.reshape(1, H))

    x_enc = enc_pair[0]
    m_enc = enc_pair[1]

    aggd1, = mp1(xsd, srcp, dstp)
    xs1d = _tc_layer1_dec(aggd1, nsd, x_enc, dec_gcn_W[0])
    aggd2, = mp1(xs1d, srcp, dstp)
    recon = _tc_layer2_dec(
        aggd2, nsd, x_enc, dec_gcn_W[1], dec_fc_W, dec_fc_b.reshape(1, -1),
        dg2, db2, dec_fc2_W, dec_fc2_b.reshape(1, H))

    loss = _tc_loss(recon, xbn_p)[0, 0]

    ppi0 = ppi_list[:, 0]
    ppi1 = ppi_list[:, 1]
    idx2 = idx.reshape(B // CH, CH)
    prodx, prodm = _build_ppi()(ppi0, ppi1, idx2, x_enc, m_enc)
    out = _tc_head(prodx, fc_out_W, fc_out_b.reshape(1, OUT))
    mout = _tc_head(prodm, fc_out_W, fc_out_b.reshape(1, OUT))

    return (out, mout, loss)


# speedup vs baseline: 1.0215x; 1.0215x over previous
import functools
import math

import jax
import jax.numpy as jnp
from jax import lax
from jax.experimental import pallas as pl
from jax.experimental.pallas import tpu as pltpu
from jax.experimental.pallas import tpu_sc as plsc

N = 10000
E = 320000
H = 128
OUT = 7
P = 500000
B = 16384
ALPHA = 0.1
EPS = 1e-5

NPAD = 10240
EPAD = 327680
CH = 128
ECH = EPAD // CH
RPT = NPAD // 16
ZR = 64

@functools.lru_cache(maxsize=None)
def _mesh():
    return plsc.VectorSubcoreMesh(core_axis_name="c", subcore_axis_name="s")


@functools.lru_cache(maxsize=None)
def _build_deg():
    return functools.partial(
        pl.kernel,
        out_type=jax.ShapeDtypeStruct((2, NPAD), jnp.float32),
        mesh=_mesh(),
        scratch_types=[
            pltpu.VMEM((ECH // 16, CH), jnp.int32),
            pltpu.VMEM((CH,), jnp.float32),
            pltpu.VMEM((RPT,), jnp.float32),
            pltpu.VMEM_SHARED((NPAD,), jnp.float32),
            pltpu.SemaphoreType.DMA,
            pltpu.SemaphoreType.DMA,
            pltpu.SemaphoreType.DMA,
            pltpu.SemaphoreType.DMA,
        ],
    )(_deg_body)


def _deg_body(eidx_hbm, out_hbm, idx_v, ones_v, stage_v, hist_sp,
              m0, m1, m2, m3):
    c = lax.axis_index("c")
    s = lax.axis_index("s")
    sems = (m0, m1, m2, m3)
    one = jnp.ones((16,), jnp.float32)
    zero = jnp.zeros((16,), jnp.float32)

    def fill(k, carry):
        ones_v[pl.ds(k * 16, 16)] = one
        return carry

    lax.fori_loop(0, CH // 16, fill, 0)

    def zstage(k, carry):
        stage_v[pl.ds(k * 16, 16)] = zero
        return carry

    lax.fori_loop(0, RPT // 16, zstage, 0)
    pltpu.sync_copy(stage_v, hist_sp.at[pl.ds(s * RPT, RPT)])
    plsc.subcore_barrier()

    nch = ECH // 16
    pltpu.sync_copy(eidx_hbm.at[c].at[pl.ds(s * nch, nch)], idx_v)

    for b in range(4):
        pltpu.async_copy(ones_v, hist_sp.at[idx_v.at[b]], sems[b],
                         add=True)

    def step(t, carry):
        j = 4 * t
        for b in range(4):
            jj4 = j + 4 + b

            @pl.when(jj4 < nch)
            def _(b=b, jj4=jj4):
                pltpu.make_async_copy(
                    ones_v, hist_sp.at[idx_v.at[jj4 - 4]], sems[b]).wait()
                pltpu.async_copy(ones_v, hist_sp.at[idx_v.at[jj4]],
                                 sems[b], add=True)
        return carry

    lax.fori_loop(0, nch // 4, step, 0)
    for b in range(4):
        pltpu.make_async_copy(
            ones_v, hist_sp.at[idx_v.at[nch - 4 + b]], sems[b]).wait()
    plsc.subcore_barrier()

    pltpu.sync_copy(hist_sp.at[pl.ds(s * RPT, RPT)], stage_v)
    pltpu.sync_copy(stage_v, out_hbm.at[c].at[pl.ds(s * RPT, RPT)])


QH = H // 4
MCH = 256
_MPN = EPAD // 16 // MCH


@functools.lru_cache(maxsize=None)
def _build_mp():
    @functools.partial(
        pl.kernel,
        out_type=jax.ShapeDtypeStruct((4, NPAD, QH), jnp.float32),
        mesh=_mesh(),
        compiler_params=pltpu.CompilerParams(use_tc_tiling_on_sc=False),
        scratch_types=[
            pltpu.VMEM((_MPN, MCH), jnp.int32),
            pltpu.VMEM((_MPN, MCH), jnp.int32),
            pltpu.VMEM((MCH, QH), jnp.float32),
            pltpu.VMEM((MCH, QH), jnp.float32),
            pltpu.VMEM((RPT, QH), jnp.float32),
            pltpu.VMEM_SHARED((NPAD, QH), jnp.float32),
            pltpu.VMEM_SHARED((NPAD, QH), jnp.float32),
            pltpu.SemaphoreType.DMA,
            pltpu.SemaphoreType.DMA,
            pltpu.SemaphoreType.DMA,
            pltpu.SemaphoreType.DMA,
        ],
    )
    def mp(xs_hbm, src_hbm, dst_hbm, out_hbm, si, di, r0, r1, zb,
           tbl, agg, g0, g1, s0, s1):
        c = lax.axis_index("c")
        s = lax.axis_index("s")
        bufs = (r0, r1)
        gsems = (g0, g1)
        ssems = (s0, s1)

        zv = jnp.zeros((16,), jnp.float32)

        base = s * _MPN
        pltpu.sync_copy(src_hbm.at[pl.ds(base, _MPN)], si)
        pltpu.sync_copy(dst_hbm.at[pl.ds(base, _MPN)], di)

        for q in range(2):
            quarter = 2 * c + q
            pltpu.sync_copy(xs_hbm.at[quarter].at[pl.ds(s * RPT, RPT)],
                            tbl.at[pl.ds(s * RPT, RPT)])

            def zs(k, carry):
                r = k // 2
                col = (k % 2) * 16
                zb[r, pl.ds(col, 16)] = zv
                return carry

            lax.fori_loop(0, RPT * 2, zs, 0)
            pltpu.sync_copy(zb, agg.at[pl.ds(s * RPT, RPT)])
            plsc.subcore_barrier()

            for b in range(2):
                pltpu.async_copy(tbl.at[si.at[b]], bufs[b], gsems[b])

            def step(t, carry):
                j = 2 * t
                for b in range(2):
                    jj = j + b
                    pltpu.make_async_copy(
                        tbl.at[si.at[jj]], bufs[b], gsems[b]).wait()
                    pltpu.async_copy(bufs[b], agg.at[di.at[jj]], ssems[b],
                                     add=True)
                for b in range(2):
                    jj2 = j + 2 + b

                    @pl.when(jj2 < _MPN)
                    def _(b=b, jj2=jj2):
                        pltpu.make_async_copy(
                            bufs[b], agg.at[di.at[jj2 - 2]],
                            ssems[b]).wait()
                        pltpu.async_copy(tbl.at[si.at[jj2]], bufs[b],
                                         gsems[b])
                return carry

            lax.fori_loop(0, _MPN // 2, step, 0)
            for b in range(2):
                pltpu.make_async_copy(
                    bufs[b], agg.at[di.at[_MPN - 2 + b]], ssems[b]).wait()
            plsc.subcore_barrier()

            off = s * RPT
            pltpu.sync_copy(agg.at[pl.ds(off, RPT)], zb)
            pltpu.sync_copy(zb, out_hbm.at[quarter].at[pl.ds(off, RPT)])
            if q == 0:
                plsc.subcore_barrier()

    return mp


_PW = B // 32
_PCH = _PW // CH


@functools.lru_cache(maxsize=None)
def _build_ppi():
    return functools.partial(
        pl.kernel,
        out_type=(
            jax.ShapeDtypeStruct((B, H), jnp.float32),
            jax.ShapeDtypeStruct((B, H), jnp.float32),
        ),
        mesh=_mesh(),
        scratch_types=[
            pltpu.VMEM((_PCH, CH), jnp.int32),
            pltpu.VMEM((_PCH, CH), jnp.int32),
            pltpu.VMEM((_PCH, CH), jnp.int32),
            pltpu.VMEM((CH, H), jnp.float32),
            pltpu.VMEM((CH, H), jnp.float32),
            pltpu.VMEM((CH, H), jnp.float32),
            pltpu.VMEM((CH, H), jnp.float32),
            pltpu.SemaphoreType.DMA,
            pltpu.SemaphoreType.DMA,
            pltpu.SemaphoreType.DMA,
            pltpu.SemaphoreType.DMA,
        ],
    )(_ppi_body)


def _ppi_body(ppi0_hbm, ppi1_hbm, idx_hbm, xenc_hbm, menc_hbm,
              outx_hbm, outm_hbm, it, n0, n1, xa, xb, ma, mb,
              m0, m1, m2, m3):
    c = lax.axis_index("c")
    s = lax.axis_index("s")
    w = c * 16 + s

    pltpu.sync_copy(idx_hbm.at[pl.ds(w * _PCH, _PCH)], it)

    def chunk(j, carry):
        pltpu.async_copy(ppi0_hbm.at[it.at[j]], n0.at[j], m0)
        pltpu.async_copy(ppi1_hbm.at[it.at[j]], n1.at[j], m1)
        pltpu.make_async_copy(ppi0_hbm.at[it.at[j]], n0.at[j], m0).wait()
        pltpu.make_async_copy(ppi1_hbm.at[it.at[j]], n1.at[j], m1).wait()

        pltpu.async_copy(xenc_hbm.at[n0.at[j]], xa, m0)
        pltpu.async_copy(xenc_hbm.at[n1.at[j]], xb, m1)
        pltpu.async_copy(menc_hbm.at[n0.at[j]], ma, m2)
        pltpu.async_copy(menc_hbm.at[n1.at[j]], mb, m3)

        def pair(ra, rb, out_hbm, sa, sb, src_a, src_b):
            pltpu.make_async_copy(src_a, ra, sa).wait()
            pltpu.make_async_copy(src_b, rb, sb).wait()

            def mul(k, cc):
                r = k // 8
                col = (k % 8) * 16
                ra[r, pl.ds(col, 16)] = (
                    ra[r, pl.ds(col, 16)] * rb[r, pl.ds(col, 16)]
                )
                return cc

            lax.fori_loop(0, CH * 8, mul, 0)
            pltpu.sync_copy(ra, out_hbm.at[pl.ds(w * _PW + j * CH, CH)])

        pair(xa, xb, outx_hbm, m0, m1,
             xenc_hbm.at[n0.at[j]], xenc_hbm.at[n1.at[j]])
        pair(ma, mb, outm_hbm, m2, m3,
             menc_hbm.at[n0.at[j]], menc_hbm.at[n1.at[j]])
        return carry

    lax.fori_loop(0, _PCH, chunk, 0)


def _tc_input(x, mask, w, b2, g2, bb2):
    def body(x_ref, m_ref, w_ref, b_ref, g_ref, bb_ref, o1_ref, o2_ref):
        h = jnp.dot(x_ref[...], w_ref[...],
                    preferred_element_type=jnp.float32)
        h = (h + b_ref[...]) * g_ref[...] + bb_ref[...]
        o1_ref[...] = h
        o2_ref[...] = h * m_ref[...]

    blk = 2000
    grid = N // blk
    return pl.pallas_call(
        body,
        grid=(grid,),
        in_specs=[
            pl.BlockSpec((blk, H), lambda i: (i, 0)),
            pl.BlockSpec((blk, H), lambda i: (i, 0)),
            pl.BlockSpec((H, H), lambda i: (0, 0)),
            pl.BlockSpec((1, H), lambda i: (0, 0)),
            pl.BlockSpec((1, H), lambda i: (0, 0)),
            pl.BlockSpec((1, H), lambda i: (0, 0)),
        ],
        out_specs=[
            pl.BlockSpec((blk, H), lambda i: (i, 0)),
            pl.BlockSpec((blk, H), lambda i: (i, 0)),
        ],
        out_shape=[
            jax.ShapeDtypeStruct((N, H), jnp.float32),
            jax.ShapeDtypeStruct((N, H), jnp.float32),
        ],
    )(x, mask, w, b2, g2, bb2)


def _split(ref, v):
    for q in range(4):
        ref[q] = v[:, q * QH:(q + 1) * QH]


def _tc_mkxs(deg, xbn_p, mx_p):
    def body(d_ref, a_ref, m_ref, xsa_ref, xsb_ref, x0_ref, nsd_ref):
        d = d_ref[...]
        nrm = jnp.where(
            d > 0.0, lax.rsqrt(jnp.maximum(d, 1e-12)), 0.0)
        nsd_ref[...] = nrm
        ns = nrm[0][:, None]
        a = a_ref[...]
        m = m_ref[...]
        _split(xsa_ref, a * ns)
        _split(xsb_ref, m * ns)
        x0_ref[0] = a
        x0_ref[1] = m

    blk = 1280
    grid = NPAD // blk
    return pl.pallas_call(
        body,
        grid=(grid,),
        in_specs=[
            pl.BlockSpec((2, blk), lambda i: (0, i)),
            pl.BlockSpec((blk, H), lambda i: (i, 0)),
            pl.BlockSpec((blk, H), lambda i: (i, 0)),
        ],
        out_specs=[
            pl.BlockSpec((4, blk, QH), lambda i: (0, i, 0)),
            pl.BlockSpec((4, blk, QH), lambda i: (0, i, 0)),
            pl.BlockSpec((2, blk, H), lambda i: (0, i, 0)),
            pl.BlockSpec((2, blk), lambda i: (0, i)),
        ],
        out_shape=[
            jax.ShapeDtypeStruct((4, NPAD, QH), jnp.float32),
            jax.ShapeDtypeStruct((4, NPAD, QH), jnp.float32),
            jax.ShapeDtypeStruct((2, NPAD, H), jnp.float32),
            jax.ShapeDtypeStruct((2, NPAD), jnp.float32),
        ],
    )(deg, xbn_p, mx_p)


def _gcn_combine(agg, nd, x0, w_ref, beta):
    feat = (1.0 - ALPHA) * (agg * nd) + ALPHA * x0
    return (1.0 - beta) * feat + beta * jnp.dot(
        feat, w_ref[...], preferred_element_type=jnp.float32)


def _join(a_ref):
    return jnp.concatenate([a_ref[0], a_ref[1], a_ref[2], a_ref[3]],
                           axis=-1)


def _tc_layer1_enc(agg_a, agg_b, nsd, x0_pair, w):
    beta = math.log(2.0)

    def body(aa_ref, ab_ref, n_ref, x0_ref, w_ref, oa_ref, ob_ref):
        nrm = n_ref[...]
        nd = nrm[1][:, None]
        ns = nrm[0][:, None]
        for a_ref, x0, o_ref in ((aa_ref, x0_ref[0], oa_ref),
                                 (ab_ref, x0_ref[1], ob_ref)):
            h = _gcn_combine(_join(a_ref), nd, x0, w_ref, beta)
            _split(o_ref, h * ns)

    blk = 1280
    grid = NPAD // blk
    return pl.pallas_call(
        body,
        grid=(grid,),
        in_specs=[
            pl.BlockSpec((4, blk, QH), lambda i: (0, i, 0)),
            pl.BlockSpec((4, blk, QH), lambda i: (0, i, 0)),
            pl.BlockSpec((2, blk), lambda i: (0, i)),
            pl.BlockSpec((2, blk, H), lambda i: (0, i, 0)),
            pl.BlockSpec((H, H), lambda i: (0, 0)),
        ],
        out_specs=[
            pl.BlockSpec((4, blk, QH), lambda i: (0, i, 0)),
            pl.BlockSpec((4, blk, QH), lambda i: (0, i, 0)),
        ],
        out_shape=[
            jax.ShapeDtypeStruct((4, NPAD, QH), jnp.float32),
            jax.ShapeDtypeStruct((4, NPAD, QH), jnp.float32),
        ],
    )(agg_a, agg_b, nsd, x0_pair, w)


def _tc_layer2_enc(agg_a, agg_b, nsd, x0_pair, w, fcw, fcb, g2, bb2,
                   fc2w, fc2b):
    beta = math.log(1.5)

    def body(aa_ref, ab_ref, n_ref, x0_ref, w_ref, fcw_ref, fcb_ref, g_ref,
             bb_ref, fc2w_ref, fc2b_ref, enc_ref, xsd_ref):
        nrm = n_ref[...]
        nd = nrm[1][:, None]
        ns = nrm[0][:, None]
        for t, a_ref in ((0, aa_ref), (1, ab_ref)):
            h = _gcn_combine(_join(a_ref), nd, x0_ref[t], w_ref, beta)
            u = jax.nn.relu(jnp.dot(h, fcw_ref[...],
                                    preferred_element_type=jnp.float32)
                            + fcb_ref[...])
            u = u * g_ref[...] + bb_ref[...]
            v = jax.nn.relu(jnp.dot(u, fc2w_ref[...],
                                    preferred_element_type=jnp.float32)
                            + fc2b_ref[...])
            enc_ref[t] = v
            if t == 0:
                _split(xsd_ref, v * ns)

    blk = 1280
    grid = NPAD // blk
    return pl.pallas_call(
        body,
        grid=(grid,),
        in_specs=[
            pl.BlockSpec((4, blk, QH), lambda i: (0, i, 0)),
            pl.BlockSpec((4, blk, QH), lambda i: (0, i, 0)),
            pl.BlockSpec((2, blk), lambda i: (0, i)),
            pl.BlockSpec((2, blk, H), lambda i: (0, i, 0)),
            pl.BlockSpec((H, H), lambda i: (0, 0)),
            pl.BlockSpec((H, 2 * H), lambda i: (0, 0)),
            pl.BlockSpec((1, 2 * H), lambda i: (0, 0)),
            pl.BlockSpec((1, 2 * H), lambda i: (0, 0)),
            pl.BlockSpec((1, 2 * H), lambda i: (0, 0)),
            pl.BlockSpec((2 * H, H), lambda i: (0, 0)),
            pl.BlockSpec((1, H), lambda i: (0, 0)),
        ],
        out_specs=[
            pl.BlockSpec((2, blk, H), lambda i: (0, i, 0)),
            pl.BlockSpec((4, blk, QH), lambda i: (0, i, 0)),
        ],
        out_shape=[
            jax.ShapeDtypeStruct((2, NPAD, H), jnp.float32),
            jax.ShapeDtypeStruct((4, NPAD, QH), jnp.float32),
        ],
    )(agg_a, agg_b, nsd, x0_pair, w, fcw, fcb, g2, bb2, fc2w, fc2b)


def _tc_layer1_dec(agg, nsd, x0, w):
    beta = math.log(2.0)

    def body(a_ref, n_ref, x0_ref, w_ref, o_ref):
        nrm = n_ref[...]
        h = _gcn_combine(_join(a_ref), nrm[1][:, None], x0_ref[...],
                         w_ref, beta)
        _split(o_ref, h * nrm[0][:, None])

    blk = 1280
    grid = NPAD // blk
    return pl.pallas_call(
        body,
        grid=(grid,),
        in_specs=[
            pl.BlockSpec((4, blk, QH), lambda i: (0, i, 0)),
            pl.BlockSpec((2, blk), lambda i: (0, i)),
            pl.BlockSpec((blk, H), lambda i: (i, 0)),
            pl.BlockSpec((H, H), lambda i: (0, 0)),
        ],
        out_specs=pl.BlockSpec((4, blk, QH), lambda i: (0, i, 0)),
        out_shape=jax.ShapeDtypeStruct((4, NPAD, QH), jnp.float32),
    )(agg, nsd, x0, w)


def _tc_layer2_dec(agg, nsd, x0, w, fcw, fcb, g2, bb2, fc2w, fc2b):
    beta = math.log(1.5)

    def body(a_ref, n_ref, x0_ref, w_ref, fcw_ref, fcb_ref, g_ref, bb_ref,
             fc2w_ref, fc2b_ref, o_ref):
        nrm = n_ref[...]
        h = _gcn_combine(_join(a_ref), nrm[1][:, None], x0_ref[...],
                         w_ref, beta)
        u = jax.nn.relu(jnp.dot(h, fcw_ref[...],
                                preferred_element_type=jnp.float32)
                        + fcb_ref[...])
        u = u * g_ref[...] + bb_ref[...]
        o_ref[...] = jax.nn.relu(jnp.dot(u, fc2w_ref[...],
                                         preferred_element_type=jnp.float32)
                                 + fc2b_ref[...])

    blk = 1280
    grid = NPAD // blk
    return pl.pallas_call(
        body,
        grid=(grid,),
        in_specs=[
            pl.BlockSpec((4, blk, QH), lambda i: (0, i, 0)),
            pl.BlockSpec((2, blk), lambda i: (0, i)),
            pl.BlockSpec((blk, H), lambda i: (i, 0)),
            pl.BlockSpec((H, H), lambda i: (0, 0)),
            pl.BlockSpec((H, 2 * H), lambda i: (0, 0)),
            pl.BlockSpec((1, 2 * H), lambda i: (0, 0)),
            pl.BlockSpec((1, 2 * H), lambda i: (0, 0)),
            pl.BlockSpec((1, 2 * H), lambda i: (0, 0)),
            pl.BlockSpec((2 * H, H), lambda i: (0, 0)),
            pl.BlockSpec((1, H), lambda i: (0, 0)),
        ],
        out_specs=pl.BlockSpec((blk, H), lambda i: (i, 0)),
        out_shape=jax.ShapeDtypeStruct((NPAD, H), jnp.float32),
    )(agg, nsd, x0, w, fcw, fcb, g2, bb2, fc2w, fc2b)


def _tc_loss(recon_p, xbn_p):
    def body(r_ref, x_ref, o_ref):
        i = pl.program_id(0)

        @pl.when(i == 0)
        def _():
            o_ref[...] = jnp.zeros((1, 1), jnp.float32)

        rows = lax.broadcasted_iota(jnp.int32, (1280, H), 0) + i * 1280
        m = (rows < N).astype(jnp.float32)
        d = (r_ref[...] - x_ref[...]) * m
        o_ref[...] += jnp.sum(d * d).reshape(1, 1) / (N * H)

    return pl.pallas_call(
        body,
        grid=(NPAD // 1280,),
        in_specs=[
            pl.BlockSpec((1280, H), lambda i: (i, 0)),
            pl.BlockSpec((1280, H), lambda i: (i, 0)),
        ],
        out_specs=pl.BlockSpec((1, 1), lambda i: (0, 0)),
        out_shape=jax.ShapeDtypeStruct((1, 1), jnp.float32),
    )(recon_p, xbn_p)


def _tc_head(prod, w, b2):
    def body(p_ref, w_ref, b_ref, o_ref):
        o_ref[...] = jnp.dot(p_ref[...], w_ref[...],
                             preferred_element_type=jnp.float32) + b_ref[...]

    blk = 2048
    return pl.pallas_call(
        body,
        grid=(B // blk,),
        in_specs=[
            pl.BlockSpec((blk, H), lambda i: (i, 0)),
            pl.BlockSpec((H, OUT), lambda i: (0, 0)),
            pl.BlockSpec((1, OUT), lambda i: (0, 0)),
        ],
        out_specs=pl.BlockSpec((blk, OUT), lambda i: (i, 0)),
        out_shape=jax.ShapeDtypeStruct((B, OUT), jnp.float32),
    )(prod, w, b2)


def kernel(x, edge_index, ppi_list, idx, mask, fc_dim_W, fc_dim_b,
           norm_in_g, norm_in_b, enc_gcn_W, enc_fc_W, enc_fc_b, enc_norm_g,
           enc_norm_b, enc_fc2_W, enc_fc2_b, dec_gcn_W, dec_fc_W, dec_fc_b,
           dec_norm_g, dec_norm_b, dec_fc2_W, dec_fc2_b, fc_out_W, fc_out_b):
    f32 = jnp.float32
    i32 = jnp.int32
    inv = 1.0 / math.sqrt(1.0 + EPS)

    src = edge_index[0]
    dst = edge_index[1]
    pad = jnp.full((EPAD - E,), N, i32)
    srcf = jnp.concatenate([src, pad])
    dstf = jnp.concatenate([dst, pad])
    eidx = jnp.stack([srcf.reshape(ECH, CH), dstf.reshape(ECH, CH)])
    srcp = srcf.reshape(EPAD // MCH, MCH)
    dstp = dstf.reshape(EPAD // MCH, MCH)

    g_in2 = (norm_in_g * inv).reshape(1, H)
    b_in2 = norm_in_b.reshape(1, H)
    fcb2 = fc_dim_b.reshape(1, H)
    eg2 = (enc_norm_g * inv).reshape(1, 2 * H)
    eb2 = enc_norm_b.reshape(1, 2 * H)
    dg2 = (dec_norm_g * inv).reshape(1, 2 * H)
    db2 = dec_norm_b.reshape(1, 2 * H)

    deg = _build_deg()(eidx)
    xbn, mx = _tc_input(x, mask, fc_dim_W, fcb2, g_in2, b_in2)
    xbn_p = jnp.pad(xbn, ((0, NPAD - N), (0, 0)))
    mx_p = jnp.pad(mx, ((0, NPAD - N), (0, 0)))

    xsa, xsb, x0_pair, nsd = _tc_mkxs(deg, xbn_p, mx_p)

    mp = _build_mp()
    agg1a = mp(xsa, srcp, dstp)
    agg1b = mp(xsb, srcp, dstp)
    xs1a, xs1b = _tc_layer1_enc(agg1a, agg1b, nsd, x0_pair, enc_gcn_W[0])
    agg2a = mp(xs1a, srcp, dstp)
    agg2b = mp(xs1b, srcp, dstp)
    enc_pair, xsd = _tc_layer2_enc(
        agg2a, agg2b, nsd, x0_pair, enc_gcn_W[1], enc_fc_W,
        enc_fc_b.reshape(1, -1), eg2, eb2, enc_fc2_W, enc_fc2_b.reshape(1, H))

    x_enc = enc_pair[0]
    m_enc = enc_pair[1]

    aggd1 = mp(xsd, srcp, dstp)
    xs1d = _tc_layer1_dec(aggd1, nsd, x_enc, dec_gcn_W[0])
    aggd2 = mp(xs1d, srcp, dstp)
    recon = _tc_layer2_dec(
        aggd2, nsd, x_enc, dec_gcn_W[1], dec_fc_W, dec_fc_b.reshape(1, -1),
        dg2, db2, dec_fc2_W, dec_fc2_b.reshape(1, H))

    loss = _tc_loss(recon, xbn_p)[0, 0]

    ppi0 = ppi_list[:, 0]
    ppi1 = ppi_list[:, 1]
    idx2 = idx.reshape(B // CH, CH)
    prodx, prodm = _build_ppi()(ppi0, ppi1, idx2, x_enc, m_enc)
    out = _tc_head(prodx, fc_out_W, fc_out_b.reshape(1, OUT))
    mout = _tc_head(prodm, fc_out_W, fc_out_b.reshape(1, OUT))

    return (out, mout, loss)
